# Initial kernel scaffold; baseline (speedup 1.0000x reference)
import jax
import jax.numpy as jnp
from jax.experimental import pallas as pl


def kernel(images, pixel_probabilities):
    raise NotImplementedError("write your pallas kernel here")


# speedup vs baseline: 414.2378x; 414.2378x over previous
import functools

import jax
import jax.numpy as jnp
from jax import lax
from jax.experimental import pallas as pl
from jax.experimental.pallas import tpu as pltpu
from jax.experimental.pallas import tpu_sc as plsc

_NUM_BINS = 256
_L = 16
_NC = 2
_NS = 16
_NW = _NC * _NS

_CHUNK = 16384
_NBUF = 2


@functools.lru_cache(maxsize=None)
def _make_lookup(n):
    per_w = n // _NW
    nch = per_w // _CHUNK
    npairs = nch // _NBUF
    assert per_w * _NW == n and npairs * _NBUF == nch

    mesh = plsc.VectorSubcoreMesh(core_axis_name="c", subcore_axis_name="s")

    @functools.partial(
        pl.kernel,
        mesh=mesh,
        out_type=jax.ShapeDtypeStruct((n,), jnp.float32),
        compiler_params=pltpu.CompilerParams(needs_layout_passes=False),
        scratch_types=[
            pltpu.VMEM((_NUM_BINS,), jnp.float32),
            pltpu.VMEM((_CHUNK,), jnp.int32),
            pltpu.VMEM((_CHUNK,), jnp.int32),
            pltpu.VMEM((_CHUNK,), jnp.float32),
            pltpu.VMEM((_CHUNK,), jnp.float32),
            pltpu.SemaphoreType.DMA,
            pltpu.SemaphoreType.DMA,
            pltpu.SemaphoreType.DMA,
            pltpu.SemaphoreType.DMA,
        ],
    )
    def lookup(idx_hbm, tab_hbm, out_hbm, tab_v, idx_v0, idx_v1,
               val_v0, val_v1, sem_in0, sem_in1, sem_out0, sem_out1):
        idx_v = (idx_v0, idx_v1)
        val_v = (val_v0, val_v1)
        sem_in = (sem_in0, sem_in1)
        sem_out = (sem_out0, sem_out1)
        wid = lax.axis_index("s") * _NC + lax.axis_index("c")
        base = wid * per_w

        pltpu.sync_copy(tab_hbm, tab_v)
        for b in range(_NBUF):
            pltpu.async_copy(idx_hbm.at[pl.ds(base + b * _CHUNK, _CHUNK)],
                             idx_v[b], sem_in[b])

        def pair(p, carry):
            for b in range(_NBUF):
                g = p * _NBUF + b
                off = base + g * _CHUNK
                pltpu.make_async_copy(idx_hbm.at[pl.ds(0, _CHUNK)],
                                      idx_v[b], sem_in[b]).wait()
                @pl.when(p > 0)
                def _():
                    pltpu.make_async_copy(val_v[b],
                                          out_hbm.at[pl.ds(0, _CHUNK)],
                                          sem_out[b]).wait()

                ib = idx_v[b]
                vb = val_v[b]

                def body(i, c):
                    sl = pl.ds(i * _L, _L)
                    vb[sl] = plsc.load_gather(tab_v, [ib[sl]])
                    return c

                lax.fori_loop(0, _CHUNK // _L, body, 0, unroll=8)

                pltpu.async_copy(vb, out_hbm.at[pl.ds(off, _CHUNK)], sem_out[b])

                @pl.when(p < npairs - 1)
                def _():
                    pltpu.async_copy(
                        idx_hbm.at[pl.ds(off + _NBUF * _CHUNK, _CHUNK)],
                        idx_v[b], sem_in[b])
            return carry

        lax.fori_loop(0, npairs, pair, 0)
        for b in range(_NBUF):
            pltpu.make_async_copy(val_v[b], out_hbm.at[pl.ds(0, _CHUNK)],
                                  sem_out[b]).wait()

    return lookup


def kernel(images, pixel_probabilities):
    flat = images.reshape(-1).astype(jnp.int32)
    out = _make_lookup(flat.shape[0])(flat, pixel_probabilities)
    return out.reshape(images.shape)


# speedup vs baseline: 952.1181x; 2.2985x over previous
import functools

import jax
import jax.numpy as jnp
from jax import lax
from jax.experimental import pallas as pl
from jax.experimental.pallas import tpu as pltpu
from jax.experimental.pallas import tpu_sc as plsc

_NUM_BINS = 256
_L = 16
_NC = 2
_NS = 16
_NW = _NC * _NS

_CHUNK = 16384
_NBUF = 2


@functools.lru_cache(maxsize=None)
def _make_lookup(n):
    per_w = n // _NW
    nch = per_w // _CHUNK
    npairs = nch // _NBUF
    assert per_w * _NW == n and npairs * _NBUF == nch

    mesh = plsc.VectorSubcoreMesh(core_axis_name="c", subcore_axis_name="s")

    @functools.partial(
        pl.kernel,
        mesh=mesh,
        out_type=jax.ShapeDtypeStruct((n,), jnp.float32),
        compiler_params=pltpu.CompilerParams(needs_layout_passes=False),
        scratch_types=[
            pltpu.VMEM((_NUM_BINS,), jnp.float32),
            pltpu.VMEM((_CHUNK,), jnp.int32),
            pltpu.VMEM((_CHUNK,), jnp.int32),
            pltpu.VMEM((_CHUNK,), jnp.float32),
            pltpu.VMEM((_CHUNK,), jnp.float32),
            pltpu.SemaphoreType.DMA,
            pltpu.SemaphoreType.DMA,
            pltpu.SemaphoreType.DMA,
            pltpu.SemaphoreType.DMA,
        ],
    )
    def lookup(idx_hbm, tab_hbm, out_hbm, tab_v, idx_v0, idx_v1,
               val_v0, val_v1, sem_in0, sem_in1, sem_out0, sem_out1):
        idx_v = (idx_v0, idx_v1)
        val_v = (val_v0, val_v1)
        sem_in = (sem_in0, sem_in1)
        sem_out = (sem_out0, sem_out1)
        wid = lax.axis_index("s") * _NC + lax.axis_index("c")
        base = wid * per_w

        pltpu.sync_copy(tab_hbm, tab_v)
        for b in range(_NBUF):
            pltpu.async_copy(idx_hbm.at[pl.ds(base + b * _CHUNK, _CHUNK)],
                             idx_v[b], sem_in[b])

        def pair(p, carry):
            for b in range(_NBUF):
                g = p * _NBUF + b
                off = base + g * _CHUNK
                pltpu.make_async_copy(idx_hbm.at[pl.ds(0, _CHUNK)],
                                      idx_v[b], sem_in[b]).wait()
                @pl.when(p > 0)
                def _():
                    pltpu.make_async_copy(val_v[b],
                                          out_hbm.at[pl.ds(0, _CHUNK)],
                                          sem_out[b]).wait()

                ib = idx_v[b]
                vb = val_v[b]

                @plsc.parallel_loop(0, _CHUNK, step=_L, unroll=8)
                def _(i):
                    sl = pl.ds(i, _L)
                    vb[sl] = plsc.load_gather(tab_v, [ib[sl]])

                pltpu.async_copy(vb, out_hbm.at[pl.ds(off, _CHUNK)], sem_out[b])

                @pl.when(p < npairs - 1)
                def _():
                    pltpu.async_copy(
                        idx_hbm.at[pl.ds(off + _NBUF * _CHUNK, _CHUNK)],
                        idx_v[b], sem_in[b])
            return carry

        lax.fori_loop(0, npairs, pair, 0)
        for b in range(_NBUF):
            pltpu.make_async_copy(val_v[b], out_hbm.at[pl.ds(0, _CHUNK)],
                                  sem_out[b]).wait()

    return lookup


def kernel(images, pixel_probabilities):
    flat = images.reshape(-1).astype(jnp.int32)
    out = _make_lookup(flat.shape[0])(flat, pixel_probabilities)
    return out.reshape(images.shape)


# speedup vs baseline: 2556.7295x; 2.6853x over previous
import functools

import jax
import jax.numpy as jnp
from jax import lax
from jax.experimental import pallas as pl
from jax.experimental.pallas import tpu as pltpu
from jax.experimental.pallas import tpu_sc as plsc

_NUM_BINS = 256
_L = 16
_NC = 2
_NS = 16
_NW = _NC * _NS

_ROWS = 32
_NBUF = 2


@functools.lru_cache(maxsize=None)
def _make_lookup(nplanes, h, w):
    per_w = nplanes // _NW
    ch_per_plane = h // _ROWS
    nch = per_w * ch_per_plane
    npairs = nch // _NBUF
    assert per_w * _NW == nplanes and npairs * _NBUF == nch

    mesh = plsc.VectorSubcoreMesh(core_axis_name="c", subcore_axis_name="s")

    @functools.partial(
        pl.kernel,
        mesh=mesh,
        out_type=jax.ShapeDtypeStruct((nplanes, h, w), jnp.float32),
        compiler_params=pltpu.CompilerParams(needs_layout_passes=False),
        scratch_types=[
            pltpu.VMEM((_NUM_BINS,), jnp.float32),
            pltpu.VMEM((_ROWS, w), jnp.int32),
            pltpu.VMEM((_ROWS, w), jnp.int32),
            pltpu.VMEM((_ROWS, w), jnp.float32),
            pltpu.VMEM((_ROWS, w), jnp.float32),
            pltpu.SemaphoreType.DMA,
            pltpu.SemaphoreType.DMA,
            pltpu.SemaphoreType.DMA,
            pltpu.SemaphoreType.DMA,
        ],
    )
    def lookup(idx_hbm, tab_hbm, out_hbm, tab_v, idx_v0, idx_v1,
               val_v0, val_v1, sem_in0, sem_in1, sem_out0, sem_out1):
        idx_v = (idx_v0, idx_v1)
        val_v = (val_v0, val_v1)
        sem_in = (sem_in0, sem_in1)
        sem_out = (sem_out0, sem_out1)
        wid = lax.axis_index("s") * _NC + lax.axis_index("c")
        pbase = wid * per_w

        def slab(t):
            plane = pbase + t // ch_per_plane
            row = (t % ch_per_plane) * _ROWS
            return plane, row

        pltpu.sync_copy(tab_hbm, tab_v)
        for b in range(_NBUF):
            plane, row = slab(b)
            pltpu.async_copy(idx_hbm.at[plane, pl.ds(row, _ROWS), :],
                             idx_v[b], sem_in[b])

        def pair(p, carry):
            for b in range(_NBUF):
                g = p * _NBUF + b
                plane, row = slab(g)
                pltpu.make_async_copy(idx_hbm.at[0, pl.ds(0, _ROWS), :],
                                      idx_v[b], sem_in[b]).wait()
                @pl.when(p > 0)
                def _():
                    pltpu.make_async_copy(val_v[b],
                                          out_hbm.at[0, pl.ds(0, _ROWS), :],
                                          sem_out[b]).wait()

                ib = idx_v[b]
                vb = val_v[b]
                per_row = w // _L

                @plsc.parallel_loop(0, _ROWS * per_row, step=1, unroll=8)
                def _(i):
                    r = i // per_row
                    cb = (i % per_row) * _L
                    sl = pl.ds(cb, _L)
                    vb[r, sl] = plsc.load_gather(tab_v, [ib[r, sl]])

                pltpu.async_copy(vb, out_hbm.at[plane, pl.ds(row, _ROWS), :],
                                 sem_out[b])

                @pl.when(p < npairs - 1)
                def _():
                    nplane, nrow = slab(g + _NBUF)
                    pltpu.async_copy(
                        idx_hbm.at[nplane, pl.ds(nrow, _ROWS), :],
                        idx_v[b], sem_in[b])
            return carry

        lax.fori_loop(0, npairs, pair, 0)
        for b in range(_NBUF):
            pltpu.make_async_copy(val_v[b], out_hbm.at[0, pl.ds(0, _ROWS), :],
                                  sem_out[b]).wait()

    return lookup


def kernel(images, pixel_probabilities):
    b, c, h, w = images.shape
    planes = images.reshape(b * c, h, w)
    out = _make_lookup(b * c, h, w)(planes, pixel_probabilities)
    return out.reshape(images.shape)
